# Initial kernel scaffold; baseline (speedup 1.0000x reference)
import jax
import jax.numpy as jnp
from jax.experimental import pallas as pl


def kernel(x, edge_index, vertex_type, node_emb, W_msg, b_msg, W_upd, b_upd, W_out):
    raise NotImplementedError("write your pallas kernel here")


# speedup vs baseline: 4.8943x; 4.8943x over previous
import functools

import jax
import jax.numpy as jnp
from jax import lax
from jax.experimental import pallas as pl
from jax.experimental.pallas import tpu as pltpu
from jax.experimental.pallas import tpu_sc as plsc

N_NODES = 10000
N_EDGES = 320000
D = 128
N_PAD = 10240
BN = 1024
CH = 128
NC, NS = 2, 16
NW = NC * NS
NCHUNK = 2528
E_PAD = NCHUNK * CH
CPW = NCHUNK // NW
RPT = N_PAD // NS


def _s1_body(x_ref, vt_ref, wax_ref, wbx_ref, ne_ref, wae_ref, wbe_ref,
             bm_ref, a_ref, b_ref):
    oh = (vt_ref[...] == lax.broadcasted_iota(jnp.int32, (1, 8), 1).astype(jnp.float32)
          ).astype(jnp.float32)
    ca = jnp.dot(ne_ref[...], wae_ref[...], preferred_element_type=jnp.float32)
    cb = jnp.dot(ne_ref[...], wbe_ref[...], preferred_element_type=jnp.float32)
    xb = x_ref[...]
    a_ref[...] = (jnp.dot(xb, wax_ref[...], preferred_element_type=jnp.float32)
                  + jnp.dot(oh, ca, preferred_element_type=jnp.float32))
    b_ref[...] = (jnp.dot(xb, wbx_ref[...], preferred_element_type=jnp.float32)
                  + jnp.dot(oh, cb, preferred_element_type=jnp.float32)
                  + bm_ref[...])


def _stage1(xp, vtf, wax, wbx, ne8, wae16, wbe16, bm):
    row = pl.BlockSpec((BN, D), lambda i: (i, 0))
    full = lambda shape: pl.BlockSpec(shape, lambda i: tuple(0 for _ in shape))
    return pl.pallas_call(
        _s1_body,
        grid=(N_PAD // BN,),
        in_specs=[row, pl.BlockSpec((BN, 1), lambda i: (i, 0)),
                  full((D, D)), full((D, D)), full((8, 16)),
                  full((16, D)), full((16, D)), full((1, D))],
        out_specs=[row, row],
        out_shape=[jax.ShapeDtypeStruct((N_PAD, D), jnp.float32),
                   jax.ShapeDtypeStruct((N_PAD, D), jnp.float32)],
    )(xp, vtf, wax, wbx, ne8, wae16, wbe16, bm)


def _sc_edge_body(a_hbm, b_hbm, src_hbm, dst_hbm, z_hbm, out_hbm,
                  src_v, dst_v, a_v, b_v, agg_sh, sem_a, sem_b):
    c = lax.axis_index("c")
    s = lax.axis_index("s")
    w = s * NC + c
    row0 = s * RPT
    pltpu.sync_copy(z_hbm.at[pl.ds(row0, RPT)], agg_sh.at[pl.ds(row0, RPT)])
    plsc.subcore_barrier()

    def chunk_body(i, carry):
        ch = w * CPW + i
        pltpu.sync_copy(src_hbm.at[ch], src_v)
        pltpu.sync_copy(dst_hbm.at[ch], dst_v)
        cpa = pltpu.async_copy(a_hbm.at[src_v], a_v, sem_a)
        cpb = pltpu.async_copy(b_hbm.at[dst_v], b_v, sem_b)
        cpa.wait()
        cpb.wait()

        def row_body(r, carry2):
            for j in range(8):
                sl = pl.ds(j * 16, 16)
                a_v[r, sl] = jnp.maximum(a_v[r, sl] + b_v[r, sl], 0.0)
            return carry2

        lax.fori_loop(0, CH, row_body, 0)
        pltpu.sync_copy(a_v, agg_sh.at[dst_v], add=True)
        return carry

    lax.fori_loop(0, CPW, chunk_body, 0)
    plsc.subcore_barrier()
    pltpu.sync_copy(agg_sh.at[pl.ds(row0, RPT)],
                    out_hbm.at[c, pl.ds(row0, RPT)])


def _stage2(a, b, src, dst, zeros):
    mesh = plsc.VectorSubcoreMesh(core_axis_name="c", subcore_axis_name="s")
    k = pl.kernel(
        _sc_edge_body,
        out_type=jax.ShapeDtypeStruct((NC, N_PAD, D), jnp.float32),
        mesh=mesh,
        scratch_types=[
            pltpu.VMEM((CH,), jnp.int32),
            pltpu.VMEM((CH,), jnp.int32),
            pltpu.VMEM((CH, D), jnp.float32),
            pltpu.VMEM((CH, D), jnp.float32),
            pltpu.VMEM_SHARED((N_PAD, D), jnp.float32),
            pltpu.SemaphoreType.DMA,
            pltpu.SemaphoreType.DMA,
        ],
    )
    return k(a, b, src, dst, zeros)


def _s3_body(x_ref, vt_ref, agg_ref, wux_ref, ne_ref, wue_ref, wua_ref,
             bu_ref, wo_ref, o_ref):
    oh = (vt_ref[...] == lax.broadcasted_iota(jnp.int32, (1, 8), 1).astype(jnp.float32)
          ).astype(jnp.float32)
    cu = jnp.dot(ne_ref[...], wue_ref[...], preferred_element_type=jnp.float32)
    agg = agg_ref[0] + agg_ref[1]
    u = (jnp.dot(x_ref[...], wux_ref[...], preferred_element_type=jnp.float32)
         + jnp.dot(oh, cu, preferred_element_type=jnp.float32)
         + jnp.dot(agg, wua_ref[...], preferred_element_type=jnp.float32)
         + bu_ref[...])
    u = jnp.maximum(u, 0.0)
    o_ref[...] = jnp.dot(u, wo_ref[...], preferred_element_type=jnp.float32)


def _stage3(xp, vtf, aggp, wux, ne8, wue16, wua, bu, wo_pad):
    row = pl.BlockSpec((BN, D), lambda i: (i, 0))
    full = lambda shape: pl.BlockSpec(shape, lambda i: tuple(0 for _ in shape))
    return pl.pallas_call(
        _s3_body,
        grid=(N_PAD // BN,),
        in_specs=[row, pl.BlockSpec((BN, 1), lambda i: (i, 0)),
                  pl.BlockSpec((NC, BN, D), lambda i: (0, i, 0)),
                  full((D, D)), full((8, 16)), full((16, D)),
                  full((D, D)), full((1, D)), full((D, D))],
        out_specs=row,
        out_shape=jax.ShapeDtypeStruct((N_PAD, D), jnp.float32),
    )(xp, vtf, aggp, wux, ne8, wue16, wua, bu, wo_pad)


def kernel(x, edge_index, vertex_type, node_emb, W_msg, b_msg, W_upd, b_upd,
           W_out):
    f32 = jnp.float32
    wax = W_msg[0:128]
    wae16 = jnp.zeros((16, D), f32).at[0:9].set(W_msg[128:137])
    wbx = W_msg[137:265]
    wbe16 = jnp.zeros((16, D), f32).at[0:9].set(W_msg[265:274])
    wux = W_upd[0:128]
    wue16 = jnp.zeros((16, D), f32).at[0:9].set(W_upd[128:137])
    wua = W_upd[137:265]
    ne8 = jnp.zeros((8, 16), f32).at[0:4, 0:9].set(node_emb)
    wo_pad = jnp.zeros((D, D), f32).at[:, 0:3].set(W_out)
    bm = (b_msg.astype(f32)).reshape(1, D)
    bu = (b_upd.astype(f32)).reshape(1, D)

    xp = jnp.zeros((N_PAD, D), f32).at[0:N_NODES].set(x)
    vtf = jnp.zeros((N_PAD, 1), f32).at[0:N_NODES, 0].set(
        vertex_type.astype(f32))
    pad_e = E_PAD - N_EDGES
    src = jnp.concatenate(
        [edge_index[0], jnp.zeros((pad_e,), jnp.int32)]).reshape(NCHUNK, CH)
    dst = jnp.concatenate(
        [edge_index[1],
         jnp.full((pad_e,), N_NODES, jnp.int32)]).reshape(NCHUNK, CH)
    zeros = jnp.zeros((N_PAD, D), f32)

    a, b = _stage1(xp, vtf, wax, wbx, ne8, wae16, wbe16, bm)
    aggp = _stage2(a, b, src, dst, zeros)
    out_full = _stage3(xp, vtf, aggp, wux, ne8, wue16, wua, bu, wo_pad)
    return out_full[0:N_NODES, 0:3]
